# baseline (speedup 1.0000x reference)
import functools

import jax
import jax.numpy as jnp
from jax import lax
from jax.experimental import pallas as pl
from jax.experimental.pallas import tpu as pltpu
from jax.experimental.pallas import tpu_sc as plsc

N = 100000
E = 3200000
NUM_GRAPHS = 64

NTILES = 32
EPT = 100352
E_PAD = NTILES * EPT
Q = 128
SB = 2048
NST = EPT // SB
FPS = SB // Q
NB = 8
NO = FPS // NB
N_PAD = 100352
RPT = N_PAD // 16


def _make_spmm(G):
    mesh = plsc.VectorSubcoreMesh(core_axis_name="c", subcore_axis_name="s")

    @functools.partial(
        pl.kernel,
        mesh=mesh,
        out_type=jax.ShapeDtypeStruct((2 * G * N_PAD, 16), jnp.float32),
        compiler_params=pltpu.CompilerParams(use_tc_tiling_on_sc=False),
        scratch_types=[
            pltpu.VMEM_SHARED((N_PAD, 16), jnp.float32),
            pltpu.VMEM((SB,), jnp.int32),
            pltpu.VMEM((SB,), jnp.int32),
            pltpu.VMEM((SB,), jnp.float32),
            *[pltpu.VMEM((Q,), jnp.int32) for _ in range(NB)],
            *[pltpu.VMEM((Q,), jnp.int32) for _ in range(NB)],
            *[pltpu.VMEM((Q, 16), jnp.float32) for _ in range(NB)],
            *[pltpu.SemaphoreType.DMA for _ in range(NB)],
            *[pltpu.SemaphoreType.DMA for _ in range(NB)],
        ],
    )
    def spmm(x_hbm, ei_hbm, vals_hbm, y_hbm, acc, row_st, col_st, val_st,
             *fire):
        c = lax.axis_index("c")
        s = lax.axis_index("s")
        wid = c * 16 + s
        base_e = wid * EPT
        col_f = fire[0:NB]
        row_f = fire[NB:2 * NB]
        rows_v = fire[2 * NB:3 * NB]
        gsem = fire[3 * NB:4 * NB]
        ssem = fire[4 * NB:5 * NB]
        rv0 = rows_v[0]
        gs0, gs1, gs2 = gsem[0], gsem[1], gsem[2]

        def build(p, off, g):
            for j in range(Q // 16):
                cc = col_st[pl.ds(off + j * 16, 16)]
                if G > 1:
                    cc = cc + g * N_PAD
                col_f[p][pl.ds(j * 16, 16)] = cc
                row_f[p][pl.ds(j * 16, 16)] = row_st[pl.ds(off + j * 16, 16)]

        def issue_g(p):
            pltpu.async_copy(x_hbm.at[col_f[p]], rows_v[p], gsem[p])

        def wait_g(p):
            pltpu.make_async_copy(x_hbm.at[col_f[p]], rows_v[p],
                                  gsem[p]).wait()

        def issue_s(p):
            pltpu.async_copy(rows_v[p], acc.at[row_f[p]], ssem[p], add=True)

        def wait_s(p):
            pltpu.make_async_copy(rows_v[p], acc.at[row_f[p]],
                                  ssem[p]).wait()

        def scale(p, off):
            for j in range(Q // 16):
                v16 = val_st[pl.ds(off + j * 16, 16)]
                for k in range(16):
                    e = j * 16 + k
                    rows_v[p][e] = rows_v[p][e] * v16[k]

        def g_body(g, carry):
            def zfill(i, zcarry):
                rv0[i] = jnp.zeros((16,), jnp.float32)
                return zcarry

            lax.fori_loop(0, Q, zfill, 0)
            for i in range(RPT // Q):
                pltpu.async_copy(rv0, acc.at[pl.ds(s * RPT + i * Q, Q)], gs0)
            for i in range(RPT // Q):
                pltpu.make_async_copy(rv0, acc.at[pl.ds(s * RPT, Q)],
                                      gs0).wait()
            plsc.subcore_barrier()

            def stage_body(st, carry2):
                eoff = base_e + st * SB
                pltpu.async_copy(ei_hbm.at[pl.ds(eoff, SB)], row_st, gs0)
                pltpu.async_copy(ei_hbm.at[pl.ds(E_PAD + eoff, SB)],
                                 col_st, gs1)
                pltpu.async_copy(vals_hbm.at[pl.ds(eoff, SB)], val_st, gs2)
                pltpu.make_async_copy(ei_hbm.at[pl.ds(eoff, SB)], row_st,
                                      gs0).wait()
                pltpu.make_async_copy(ei_hbm.at[pl.ds(eoff, SB)], col_st,
                                      gs1).wait()
                pltpu.make_async_copy(vals_hbm.at[pl.ds(eoff, SB)], val_st,
                                      gs2).wait()

                for f in range(NB - 2):
                    build(f, f * Q, g)
                    issue_g(f)

                def octet(i, qcarry):
                    qbase = i * NB * Q
                    for p in range(NB):
                        foff = qbase + p * Q
                        wait_g(p)
                        scale(p, foff)
                        issue_s(p)
                        q2 = (p + NB - 2) % NB
                        if p < 2:
                            @pl.when(i > 0)
                            def _():
                                wait_s(q2)
                            build(q2, foff + (NB - 2) * Q, g)
                            issue_g(q2)
                        else:
                            wait_s(q2)

                            @pl.when(i < NO - 1)
                            def _():
                                build(q2, foff + (NB - 2) * Q, g)
                                issue_g(q2)
                    return qcarry

                lax.fori_loop(0, NO, octet, 0)
                wait_s(NB - 2)
                wait_s(NB - 1)
                return carry2

            lax.fori_loop(0, NST, stage_body, 0)
            plsc.subcore_barrier()
            out_off = c * (G * N_PAD) + g * N_PAD + s * RPT
            pltpu.sync_copy(acc.at[pl.ds(s * RPT, RPT)],
                            y_hbm.at[pl.ds(out_off, RPT)])
            return carry

        lax.fori_loop(0, G, g_body, 0)

    return spmm


_BN = 1792


def _dense1(y0, w1p, b1):
    def body(y_ref, w_ref, b_ref, o_ref):
        h = y_ref[0] + y_ref[1]
        z = jnp.dot(h, w_ref[...], preferred_element_type=jnp.float32)
        r = jnp.maximum(z + b_ref[...], 0.0)
        for g in range(4):
            o_ref[g] = r[:, g * 16:(g + 1) * 16]

    return pl.pallas_call(
        body,
        grid=(N_PAD // _BN,),
        in_specs=[
            pl.BlockSpec((2, _BN, 16), lambda i: (0, i, 0)),
            pl.BlockSpec((16, 64), lambda i: (0, 0)),
            pl.BlockSpec((1, 64), lambda i: (0, 0)),
        ],
        out_specs=pl.BlockSpec((4, _BN, 16), lambda i: (0, i, 0)),
        out_shape=jax.ShapeDtypeStruct((4, N_PAD, 16), jnp.float32),
    )(y0, w1p, b1)


def _dense2(y, wr, b):
    def body(y_ref, w_ref, b_ref, o_ref):
        z = jnp.zeros((_BN, 64), jnp.float32)
        for g in range(4):
            h = y_ref[0, g] + y_ref[1, g]
            z = z + jnp.dot(h, w_ref[g], preferred_element_type=jnp.float32)
        r = jnp.maximum(z + b_ref[...], 0.0)
        for g in range(4):
            o_ref[g] = r[:, g * 16:(g + 1) * 16]

    return pl.pallas_call(
        body,
        grid=(N_PAD // _BN,),
        in_specs=[
            pl.BlockSpec((2, 4, _BN, 16), lambda i: (0, 0, i, 0)),
            pl.BlockSpec((4, 16, 64), lambda i: (0, 0, 0)),
            pl.BlockSpec((1, 64), lambda i: (0, 0)),
        ],
        out_specs=pl.BlockSpec((4, _BN, 16), lambda i: (0, i, 0)),
        out_shape=jax.ShapeDtypeStruct((4, N_PAD, 16), jnp.float32),
    )(y, wr, b)


def _dense3_pool(y, wr, b, x1s, x2s, batch):
    def body(y_ref, w_ref, b_ref, x1_ref, x2_ref, bt_ref, sums_ref, cnt_ref):
        i = pl.program_id(0)
        z = jnp.zeros((_BN, 64), jnp.float32)
        for g in range(4):
            h = y_ref[0, g] + y_ref[1, g]
            z = z + jnp.dot(h, w_ref[g], preferred_element_type=jnp.float32)
        x3 = jnp.maximum(z + b_ref[...], 0.0)
        oh = (bt_ref[0].reshape(_BN, 1)
              == lax.broadcasted_iota(jnp.int32, (1, NUM_GRAPHS), 1))
        oh = oh.astype(jnp.float32)

        @pl.when(i == 0)
        def _():
            sums_ref[...] = jnp.zeros_like(sums_ref)
            cnt_ref[...] = jnp.zeros_like(cnt_ref)

        cnt_ref[...] += jnp.sum(oh, axis=0, keepdims=True)
        for g in range(4):
            xm = (x1_ref[g] + x2_ref[g] + x3[:, g * 16:(g + 1) * 16]) * (1.0 / 3.0)
            sums_ref[g] += lax.dot_general(
                oh, xm, (((0,), (0,)), ((), ())),
                preferred_element_type=jnp.float32)

    return pl.pallas_call(
        body,
        grid=(N_PAD // _BN,),
        in_specs=[
            pl.BlockSpec((2, 4, _BN, 16), lambda i: (0, 0, i, 0)),
            pl.BlockSpec((4, 16, 64), lambda i: (0, 0, 0)),
            pl.BlockSpec((1, 64), lambda i: (0, 0)),
            pl.BlockSpec((4, _BN, 16), lambda i: (0, i, 0)),
            pl.BlockSpec((4, _BN, 16), lambda i: (0, i, 0)),
            pl.BlockSpec((1, 1, _BN), lambda i: (i, 0, 0)),
        ],
        out_specs=[
            pl.BlockSpec((4, NUM_GRAPHS, 16), lambda i: (0, 0, 0)),
            pl.BlockSpec((1, NUM_GRAPHS), lambda i: (0, 0)),
        ],
        out_shape=[
            jax.ShapeDtypeStruct((4, NUM_GRAPHS, 16), jnp.float32),
            jax.ShapeDtypeStruct((1, NUM_GRAPHS), jnp.float32),
        ],
    )(y, wr, b, x1s, x2s, batch)


def _head(sums, counts, wlr, bl):
    def body(s_ref, c_ref, w_ref, b_ref, o_ref):
        cnt = jnp.maximum(c_ref[0, :], 1.0).reshape(NUM_GRAPHS, 1)
        z = jnp.zeros((NUM_GRAPHS, 10), jnp.float32)
        for g in range(4):
            z = z + jnp.dot(s_ref[g] / cnt, w_ref[g],
                            preferred_element_type=jnp.float32)
        z = z + b_ref[...]
        m = jnp.max(z, axis=1, keepdims=True)
        e = jnp.exp(z - m)
        o_ref[...] = e / jnp.sum(e, axis=1, keepdims=True)

    return pl.pallas_call(
        body,
        in_specs=[
            pl.BlockSpec((4, NUM_GRAPHS, 16), lambda: (0, 0, 0)),
            pl.BlockSpec((1, NUM_GRAPHS), lambda: (0, 0)),
            pl.BlockSpec((4, 16, 10), lambda: (0, 0, 0)),
            pl.BlockSpec((1, 10), lambda: (0, 0)),
        ],
        out_specs=pl.BlockSpec((NUM_GRAPHS, 10), lambda: (0, 0)),
        out_shape=jax.ShapeDtypeStruct((NUM_GRAPHS, 10), jnp.float32),
    )(sums, counts, wlr, bl)


def kernel(X, L_indices, L_values, batch, W1, b1, W2, b2, W3, b3, Wl, bl):
    pad = E_PAD - E
    ei = jnp.pad(L_indices, ((0, 0), (0, pad))).reshape(2 * E_PAD)
    vals_p = jnp.pad(L_values, (0, pad))

    x16 = jnp.pad(X[0], ((0, N_PAD - N), (0, 11)))
    w1p = jnp.pad(W1, ((0, 11), (0, 0)))
    batch_p = jnp.pad(batch[0], (0, N_PAD - N), constant_values=NUM_GRAPHS)

    spmm1 = _make_spmm(1)
    spmm4 = _make_spmm(4)

    y0 = spmm1(x16, ei, vals_p).reshape(2, N_PAD, 16)
    x1s = _dense1(y0, w1p, b1.reshape(1, 64))

    y1 = spmm4(x1s.reshape(4 * N_PAD, 16), ei, vals_p)
    x2s = _dense2(y1.reshape(2, 4, N_PAD, 16), W2.reshape(4, 16, 64),
                  b2.reshape(1, 64))

    y2 = spmm4(x2s.reshape(4 * N_PAD, 16), ei, vals_p)
    sums, counts = _dense3_pool(y2.reshape(2, 4, N_PAD, 16),
                                W3.reshape(4, 16, 64), b3.reshape(1, 64),
                                x1s, x2s,
                                batch_p.reshape(N_PAD // _BN, 1, _BN))

    return _head(sums, counts, Wl.reshape(4, 16, 10), bl.reshape(1, 10))


# speedup vs baseline: 1.0986x; 1.0986x over previous
import functools

import jax
import jax.numpy as jnp
from jax import lax
from jax.experimental import pallas as pl
from jax.experimental.pallas import tpu as pltpu
from jax.experimental.pallas import tpu_sc as plsc

N = 100000
E = 3200000
NUM_GRAPHS = 64

NTILES = 32
EPT = 100352
E_PAD = NTILES * EPT
Q = 128
SB = 3584
NST = EPT // SB
FPS = SB // Q
NB = 7
NO = FPS // NB
N_PAD = 100352
RPT = N_PAD // 16


def _make_spmm(G):
    mesh = plsc.VectorSubcoreMesh(core_axis_name="c", subcore_axis_name="s")

    @functools.partial(
        pl.kernel,
        mesh=mesh,
        out_type=jax.ShapeDtypeStruct((2 * G * N_PAD, 16), jnp.float32),
        compiler_params=pltpu.CompilerParams(use_tc_tiling_on_sc=False),
        scratch_types=[
            pltpu.VMEM_SHARED((N_PAD, 16), jnp.float32),
            pltpu.VMEM((SB,), jnp.int32),
            pltpu.VMEM((SB,), jnp.int32),
            pltpu.VMEM((SB,), jnp.float32),
            *[pltpu.VMEM((Q,), jnp.int32) for _ in range(NB)],
            *[pltpu.VMEM((Q,), jnp.int32) for _ in range(NB)],
            *[pltpu.VMEM((Q, 16), jnp.float32) for _ in range(NB)],
            *[pltpu.SemaphoreType.DMA for _ in range(NB)],
            *[pltpu.SemaphoreType.DMA for _ in range(NB)],
        ],
    )
    def spmm(x_hbm, ei_hbm, vals_hbm, y_hbm, acc, row_st, col_st, val_st,
             *fire):
        c = lax.axis_index("c")
        s = lax.axis_index("s")
        wid = c * 16 + s
        base_e = wid * EPT
        col_f = fire[0:NB]
        row_f = fire[NB:2 * NB]
        rows_v = fire[2 * NB:3 * NB]
        gsem = fire[3 * NB:4 * NB]
        ssem = fire[4 * NB:5 * NB]
        rv0 = rows_v[0]
        gs0, gs1, gs2 = gsem[0], gsem[1], gsem[2]

        def build(p, off, g):
            for j in range(Q // 16):
                cc = col_st[pl.ds(off + j * 16, 16)]
                if G > 1:
                    cc = cc + g * N_PAD
                col_f[p][pl.ds(j * 16, 16)] = cc
                row_f[p][pl.ds(j * 16, 16)] = row_st[pl.ds(off + j * 16, 16)]

        def issue_g(p):
            pltpu.async_copy(x_hbm.at[col_f[p]], rows_v[p], gsem[p])

        def wait_g(p):
            pltpu.make_async_copy(x_hbm.at[col_f[p]], rows_v[p],
                                  gsem[p]).wait()

        def issue_s(p):
            pltpu.async_copy(rows_v[p], acc.at[row_f[p]], ssem[p], add=True)

        def wait_s(p):
            pltpu.make_async_copy(rows_v[p], acc.at[row_f[p]],
                                  ssem[p]).wait()

        def scale(p, off):
            for j in range(Q // 16):
                v16 = val_st[pl.ds(off + j * 16, 16)]
                for k in range(16):
                    e = j * 16 + k
                    rows_v[p][e] = rows_v[p][e] * v16[k]

        def g_body(g, carry):
            def zfill(i, zcarry):
                rv0[i] = jnp.zeros((16,), jnp.float32)
                return zcarry

            lax.fori_loop(0, Q, zfill, 0)
            for i in range(RPT // Q):
                pltpu.async_copy(rv0, acc.at[pl.ds(s * RPT + i * Q, Q)], gs0)
            for i in range(RPT // Q):
                pltpu.make_async_copy(rv0, acc.at[pl.ds(s * RPT, Q)],
                                      gs0).wait()
            plsc.subcore_barrier()

            def stage_body(st, carry2):
                eoff = base_e + st * SB
                pltpu.async_copy(ei_hbm.at[pl.ds(eoff, SB)], row_st, gs0)
                pltpu.async_copy(ei_hbm.at[pl.ds(E_PAD + eoff, SB)],
                                 col_st, gs1)
                pltpu.async_copy(vals_hbm.at[pl.ds(eoff, SB)], val_st, gs2)
                pltpu.make_async_copy(ei_hbm.at[pl.ds(eoff, SB)], row_st,
                                      gs0).wait()
                pltpu.make_async_copy(ei_hbm.at[pl.ds(eoff, SB)], col_st,
                                      gs1).wait()
                pltpu.make_async_copy(vals_hbm.at[pl.ds(eoff, SB)], val_st,
                                      gs2).wait()

                for f in range(NB - 2):
                    build(f, f * Q, g)
                    issue_g(f)

                def octet(i, qcarry):
                    qbase = i * NB * Q
                    for p in range(NB):
                        foff = qbase + p * Q
                        wait_g(p)
                        scale(p, foff)
                        issue_s(p)
                        q2 = (p + NB - 2) % NB
                        if p < 2:
                            @pl.when(i > 0)
                            def _():
                                wait_s(q2)
                            build(q2, foff + (NB - 2) * Q, g)
                            issue_g(q2)
                        else:
                            wait_s(q2)

                            @pl.when(i < NO - 1)
                            def _():
                                build(q2, foff + (NB - 2) * Q, g)
                                issue_g(q2)
                    return qcarry

                lax.fori_loop(0, NO, octet, 0)
                wait_s(NB - 2)
                wait_s(NB - 1)
                return carry2

            lax.fori_loop(0, NST, stage_body, 0)
            plsc.subcore_barrier()
            out_off = c * (G * N_PAD) + g * N_PAD + s * RPT
            pltpu.sync_copy(acc.at[pl.ds(s * RPT, RPT)],
                            y_hbm.at[pl.ds(out_off, RPT)])
            return carry

        lax.fori_loop(0, G, g_body, 0)

    return spmm


_BN = 1792


def _dense1(y0, w1p, b1):
    def body(y_ref, w_ref, b_ref, o_ref):
        h = y_ref[0] + y_ref[1]
        z = jnp.dot(h, w_ref[...], preferred_element_type=jnp.float32)
        r = jnp.maximum(z + b_ref[...], 0.0)
        for g in range(4):
            o_ref[g] = r[:, g * 16:(g + 1) * 16]

    return pl.pallas_call(
        body,
        grid=(N_PAD // _BN,),
        in_specs=[
            pl.BlockSpec((2, _BN, 16), lambda i: (0, i, 0)),
            pl.BlockSpec((16, 64), lambda i: (0, 0)),
            pl.BlockSpec((1, 64), lambda i: (0, 0)),
        ],
        out_specs=pl.BlockSpec((4, _BN, 16), lambda i: (0, i, 0)),
        out_shape=jax.ShapeDtypeStruct((4, N_PAD, 16), jnp.float32),
    )(y0, w1p, b1)


def _dense2(y, wr, b):
    def body(y_ref, w_ref, b_ref, o_ref):
        z = jnp.zeros((_BN, 64), jnp.float32)
        for g in range(4):
            h = y_ref[0, g] + y_ref[1, g]
            z = z + jnp.dot(h, w_ref[g], preferred_element_type=jnp.float32)
        r = jnp.maximum(z + b_ref[...], 0.0)
        for g in range(4):
            o_ref[g] = r[:, g * 16:(g + 1) * 16]

    return pl.pallas_call(
        body,
        grid=(N_PAD // _BN,),
        in_specs=[
            pl.BlockSpec((2, 4, _BN, 16), lambda i: (0, 0, i, 0)),
            pl.BlockSpec((4, 16, 64), lambda i: (0, 0, 0)),
            pl.BlockSpec((1, 64), lambda i: (0, 0)),
        ],
        out_specs=pl.BlockSpec((4, _BN, 16), lambda i: (0, i, 0)),
        out_shape=jax.ShapeDtypeStruct((4, N_PAD, 16), jnp.float32),
    )(y, wr, b)


def _dense3_pool(y, wr, b, x1s, x2s, batch):
    def body(y_ref, w_ref, b_ref, x1_ref, x2_ref, bt_ref, sums_ref, cnt_ref):
        i = pl.program_id(0)
        z = jnp.zeros((_BN, 64), jnp.float32)
        for g in range(4):
            h = y_ref[0, g] + y_ref[1, g]
            z = z + jnp.dot(h, w_ref[g], preferred_element_type=jnp.float32)
        x3 = jnp.maximum(z + b_ref[...], 0.0)
        oh = (bt_ref[0].reshape(_BN, 1)
              == lax.broadcasted_iota(jnp.int32, (1, NUM_GRAPHS), 1))
        oh = oh.astype(jnp.float32)

        @pl.when(i == 0)
        def _():
            sums_ref[...] = jnp.zeros_like(sums_ref)
            cnt_ref[...] = jnp.zeros_like(cnt_ref)

        cnt_ref[...] += jnp.sum(oh, axis=0, keepdims=True)
        for g in range(4):
            xm = (x1_ref[g] + x2_ref[g] + x3[:, g * 16:(g + 1) * 16]) * (1.0 / 3.0)
            sums_ref[g] += lax.dot_general(
                oh, xm, (((0,), (0,)), ((), ())),
                preferred_element_type=jnp.float32)

    return pl.pallas_call(
        body,
        grid=(N_PAD // _BN,),
        in_specs=[
            pl.BlockSpec((2, 4, _BN, 16), lambda i: (0, 0, i, 0)),
            pl.BlockSpec((4, 16, 64), lambda i: (0, 0, 0)),
            pl.BlockSpec((1, 64), lambda i: (0, 0)),
            pl.BlockSpec((4, _BN, 16), lambda i: (0, i, 0)),
            pl.BlockSpec((4, _BN, 16), lambda i: (0, i, 0)),
            pl.BlockSpec((1, 1, _BN), lambda i: (i, 0, 0)),
        ],
        out_specs=[
            pl.BlockSpec((4, NUM_GRAPHS, 16), lambda i: (0, 0, 0)),
            pl.BlockSpec((1, NUM_GRAPHS), lambda i: (0, 0)),
        ],
        out_shape=[
            jax.ShapeDtypeStruct((4, NUM_GRAPHS, 16), jnp.float32),
            jax.ShapeDtypeStruct((1, NUM_GRAPHS), jnp.float32),
        ],
    )(y, wr, b, x1s, x2s, batch)


def _head(sums, counts, wlr, bl):
    def body(s_ref, c_ref, w_ref, b_ref, o_ref):
        cnt = jnp.maximum(c_ref[0, :], 1.0).reshape(NUM_GRAPHS, 1)
        z = jnp.zeros((NUM_GRAPHS, 10), jnp.float32)
        for g in range(4):
            z = z + jnp.dot(s_ref[g] / cnt, w_ref[g],
                            preferred_element_type=jnp.float32)
        z = z + b_ref[...]
        m = jnp.max(z, axis=1, keepdims=True)
        e = jnp.exp(z - m)
        o_ref[...] = e / jnp.sum(e, axis=1, keepdims=True)

    return pl.pallas_call(
        body,
        in_specs=[
            pl.BlockSpec((4, NUM_GRAPHS, 16), lambda: (0, 0, 0)),
            pl.BlockSpec((1, NUM_GRAPHS), lambda: (0, 0)),
            pl.BlockSpec((4, 16, 10), lambda: (0, 0, 0)),
            pl.BlockSpec((1, 10), lambda: (0, 0)),
        ],
        out_specs=pl.BlockSpec((NUM_GRAPHS, 10), lambda: (0, 0)),
        out_shape=jax.ShapeDtypeStruct((NUM_GRAPHS, 10), jnp.float32),
    )(sums, counts, wlr, bl)


def kernel(X, L_indices, L_values, batch, W1, b1, W2, b2, W3, b3, Wl, bl):
    pad = E_PAD - E
    ei = jnp.pad(L_indices, ((0, 0), (0, pad))).reshape(2 * E_PAD)
    vals_p = jnp.pad(L_values, (0, pad))

    x16 = jnp.pad(X[0], ((0, N_PAD - N), (0, 11)))
    w1p = jnp.pad(W1, ((0, 11), (0, 0)))
    batch_p = jnp.pad(batch[0], (0, N_PAD - N), constant_values=NUM_GRAPHS)

    spmm1 = _make_spmm(1)
    spmm4 = _make_spmm(4)

    y0 = spmm1(x16, ei, vals_p).reshape(2, N_PAD, 16)
    x1s = _dense1(y0, w1p, b1.reshape(1, 64))

    y1 = spmm4(x1s.reshape(4 * N_PAD, 16), ei, vals_p)
    x2s = _dense2(y1.reshape(2, 4, N_PAD, 16), W2.reshape(4, 16, 64),
                  b2.reshape(1, 64))

    y2 = spmm4(x2s.reshape(4 * N_PAD, 16), ei, vals_p)
    sums, counts = _dense3_pool(y2.reshape(2, 4, N_PAD, 16),
                                W3.reshape(4, 16, 64), b3.reshape(1, 64),
                                x1s, x2s,
                                batch_p.reshape(N_PAD // _BN, 1, _BN))

    return _head(sums, counts, Wl.reshape(4, 16, 10), bl.reshape(1, 10))
